# baseline (speedup 1.0000x reference)
import functools

import jax
import jax.numpy as jnp
from jax import lax
from jax.experimental import pallas as pl
from jax.experimental.pallas import tpu as pltpu
from jax.experimental.pallas import tpu_sc as plsc

NN = 10000
EE = 320000
DD = 128
BB = 4096

NC, NS, L = 2, 16, 16
NW = NC * NS
RPT = 320
ACC_ROWS = RPT + 1
FMIN = -3.4028235e38
CHUNK = 3200
GROUPS = CHUNK // L
GB = 64
NGRP = GB // L
JUNK = CHUNK + GB
SELSZ = CHUNK + GB + L

_mesh = plsc.VectorSubcoreMesh(core_axis_name="c", subcore_axis_name="s")


def _worker_id():
    return lax.axis_index("s") * NC + lax.axis_index("c")


def _segmax_body(feat_hbm, src_hbm, dst_hbm, out_hbm,
                 acc_v, srcc_v, dstc_v, selsrc_v, seldst_v, rows0_v, rows1_v,
                 sem0, sem1):
    wid = _worker_id()
    lo = jnp.minimum(wid * RPT, NN - RPT)
    hi = lo + RPT

    def init_row(r, carry):
        for k in range(DD // L):
            acc_v[r, pl.ds(k * L, L)] = jnp.full((L,), FMIN, jnp.float32)
        return carry

    lax.fori_loop(0, ACC_ROWS, init_row, 0)

    def chunk_body(ci, carry):
        base = ci * CHUNK
        pltpu.sync_copy(src_hbm.at[pl.ds(base, CHUNK)], srcc_v)
        pltpu.sync_copy(dst_hbm.at[pl.ds(base, CHUNK)], dstc_v)

        def sel_body(g, cnt):
            vs = srcc_v[pl.ds(g * L, L)]
            vd = dstc_v[pl.ds(g * L, L)]
            m = (vd >= lo) & (vd < hi)
            cum = plsc.cumsum(m.astype(jnp.int32))
            pos = jnp.where(m, cnt + cum - 1, JUNK)
            plsc.store_scatter(selsrc_v, [pos], vs)
            plsc.store_scatter(seldst_v, [pos], vd - lo)
            return cnt + cum[L - 1]

        cnt = lax.fori_loop(0, GROUPS, sel_body, jnp.int32(0))

        for t in range(NGRP):
            selsrc_v[pl.ds(cnt + t * L, L)] = jnp.zeros((L,), jnp.int32)
            seldst_v[pl.ds(cnt + t * L, L)] = jnp.full((L,), RPT, jnp.int32)
        ngroups = (cnt + (GB - 1)) // GB

        def fire(b, buf, sem):
            pltpu.async_copy(
                feat_hbm.at[selsrc_v.at[pl.ds(b * GB, GB)]], buf, sem)

        def wait(b, buf, sem):
            pltpu.make_async_copy(
                feat_hbm.at[selsrc_v.at[pl.ds(b * GB, GB)]], buf, sem).wait()

        def process(b, buf):
            for g in range(NGRP):
                ldv = seldst_v[pl.ds(b * GB + g * L, L)]
                for j in range(L):
                    ld = ldv[j]
                    for k in range(DD // L):
                        sl = pl.ds(k * L, L)
                        acc_v[ld, sl] = jnp.maximum(
                            acc_v[ld, sl], buf[g * L + j, sl])

        @pl.when(ngroups > 0)
        def _():
            fire(0, rows0_v, sem0)

        def grp_body(b, carry):
            par = lax.rem(b, 2)
            nxt = b + 1 < ngroups

            @pl.when(nxt & (par == 0))
            def _():
                fire(b + 1, rows1_v, sem1)

            @pl.when(nxt & (par == 1))
            def _():
                fire(b + 1, rows0_v, sem0)

            @pl.when(par == 0)
            def _():
                wait(b, rows0_v, sem0)
                process(b, rows0_v)

            @pl.when(par == 1)
            def _():
                wait(b, rows1_v, sem1)
                process(b, rows1_v)

            return carry

        lax.fori_loop(0, ngroups, grp_body, 0)
        return carry

    lax.fori_loop(0, EE // CHUNK, chunk_body, 0)

    def fin_row(r, carry):
        for k in range(DD // L):
            sl = pl.ds(k * L, L)
            v = acc_v[r, sl]
            acc_v[r, sl] = jnp.where(v == FMIN, 0.0, v)
        return carry

    lax.fori_loop(0, RPT, fin_row, 0)
    pltpu.sync_copy(acc_v.at[pl.ds(0, RPT)], out_hbm.at[pl.ds(lo, RPT)])


_segmax = pl.kernel(
    _segmax_body,
    out_type=jax.ShapeDtypeStruct((NN, DD), jnp.float32),
    mesh=_mesh,
    compiler_params=pltpu.CompilerParams(needs_layout_passes=False),
    scratch_types=[
        pltpu.VMEM((ACC_ROWS, DD), jnp.float32),
        pltpu.VMEM((CHUNK,), jnp.int32),
        pltpu.VMEM((CHUNK,), jnp.int32),
        pltpu.VMEM((SELSZ,), jnp.int32),
        pltpu.VMEM((SELSZ,), jnp.int32),
        pltpu.VMEM((GB, DD), jnp.float32),
        pltpu.VMEM((GB, DD), jnp.float32),
        pltpu.SemaphoreType.DMA,
        pltpu.SemaphoreType.DMA,
    ],
)

BPW = BB // NW


def _gather2_body(h1_hbm, agg2_hbm, batch_hbm, outh_hbm, outa_hbm,
                  idx_v, rows_v, sem):
    wid = _worker_id()
    base = wid * BPW
    pltpu.sync_copy(batch_hbm.at[pl.ds(base, BPW)], idx_v)
    pltpu.async_copy(h1_hbm.at[idx_v], rows_v, sem).wait()
    pltpu.sync_copy(rows_v, outh_hbm.at[pl.ds(base, BPW)])
    pltpu.async_copy(agg2_hbm.at[idx_v], rows_v, sem).wait()
    pltpu.sync_copy(rows_v, outa_hbm.at[pl.ds(base, BPW)])


_gather2 = pl.kernel(
    _gather2_body,
    out_type=(
        jax.ShapeDtypeStruct((BB, DD), jnp.float32),
        jax.ShapeDtypeStruct((BB, DD), jnp.float32),
    ),
    mesh=_mesh,
    compiler_params=pltpu.CompilerParams(needs_layout_passes=False),
    scratch_types=[
        pltpu.VMEM((BPW,), jnp.int32),
        pltpu.VMEM((BPW, DD), jnp.float32),
        pltpu.SemaphoreType.DMA,
    ],
)


def _l1_body(x_ref, agg_ref, w1_ref, b1_ref, o_ref):
    w = w1_ref[...]
    h = lax.dot_general(x_ref[...], w[:, :DD], (((1,), (1,)), ((), ())),
                        preferred_element_type=jnp.float32)
    h += lax.dot_general(agg_ref[...], w[:, DD:], (((1,), (1,)), ((), ())),
                         preferred_element_type=jnp.float32)
    o_ref[...] = jnp.maximum(h + b1_ref[...], 0.0)


L1_BLK = 1000

_layer1 = pl.pallas_call(
    _l1_body,
    grid=(NN // L1_BLK,),
    in_specs=[
        pl.BlockSpec((L1_BLK, DD), lambda i: (i, 0)),
        pl.BlockSpec((L1_BLK, DD), lambda i: (i, 0)),
        pl.BlockSpec((DD, 2 * DD), lambda i: (0, 0)),
        pl.BlockSpec((1, DD), lambda i: (0, 0)),
    ],
    out_specs=pl.BlockSpec((L1_BLK, DD), lambda i: (i, 0)),
    out_shape=jax.ShapeDtypeStruct((NN, DD), jnp.float32),
)


def _l2_body(hb_ref, ab_ref, w2_ref, b2_ref, wo_ref, bo_ref, o_ref):
    wo = wo_ref[...]
    weff = lax.dot_general(wo, w2_ref[...], (((1,), (0,)), ((), ())),
                           preferred_element_type=jnp.float32)
    h = lax.dot_general(hb_ref[...], weff[:, :DD], (((1,), (1,)), ((), ())),
                        preferred_element_type=jnp.float32)
    h += lax.dot_general(ab_ref[...], weff[:, DD:], (((1,), (1,)), ((), ())),
                         preferred_element_type=jnp.float32)
    beff = lax.dot_general(b2_ref[...], wo, (((1,), (1,)), ((), ())),
                           preferred_element_type=jnp.float32)
    o_ref[...] = h + beff + bo_ref[...]


_layer2 = pl.pallas_call(
    _l2_body,
    out_shape=jax.ShapeDtypeStruct((BB, DD), jnp.float32),
)


@jax.jit
def kernel(x, edge_index, batch, W1, b1, W2, b2, Wout, bout):
    src = edge_index[0]
    dst = edge_index[1]
    agg1 = _segmax(x, src, dst)
    h1 = _layer1(x, agg1, W1, b1.reshape(1, DD))
    agg2 = _segmax(h1, src, dst)
    h1b, a2b = _gather2(h1, agg2, batch)
    return _layer2(h1b, a2b, W2, b2.reshape(1, DD), Wout,
                   bout.reshape(1, DD))


# speedup vs baseline: 3.0230x; 3.0230x over previous
import functools

import jax
import jax.numpy as jnp
from jax import lax
from jax.experimental import pallas as pl
from jax.experimental.pallas import tpu as pltpu
from jax.experimental.pallas import tpu_sc as plsc

NN = 10000
EE = 320000
DD = 128
BB = 4096

NC, NS, L = 2, 16, 16
NW = NC * NS
RPT = 320
ACC_ROWS = RPT + 1
FMIN = -3.4028235e38
CHUNK = 3200
GROUPS = CHUNK // L
GB = 16
NGRP = GB // L
JUNK = CHUNK + GB
SELSZ = CHUNK + GB + L

_mesh = plsc.VectorSubcoreMesh(core_axis_name="c", subcore_axis_name="s")


def _worker_id():
    return lax.axis_index("s") * NC + lax.axis_index("c")


FJUNK = RPT + 8
FLAGSZ = RPT + 32


def _segmax_impl(filtered, feat_hbm, src_hbm, dst_hbm, batch_hbm, out_hbm,
                 acc_v, srcc_v, dstc_v, selsrc_v, seldst_v, rows0_v, rows1_v,
                 batch_v, flags_v, sem0, sem1):
    wid = _worker_id()
    lo = jnp.minimum(wid * RPT, NN - RPT)

    def init_row(r, carry):
        for k in range(DD // L):
            acc_v[r, pl.ds(k * L, L)] = jnp.full((L,), FMIN, jnp.float32)
        return carry

    lax.fori_loop(0, ACC_ROWS, init_row, 0)

    if filtered:
        def zf(i, carry):
            flags_v[pl.ds(i * L, L)] = jnp.zeros((L,), jnp.int32)
            return carry

        lax.fori_loop(0, FLAGSZ // L, zf, 0)
        pltpu.sync_copy(batch_hbm, batch_v)
        ones = jnp.full((L,), 1, jnp.int32)

        def bscan(g, carry):
            bv = batch_v[pl.ds(g * L, L)]
            rel = bv - lo
            inr = (rel >= 0) & (rel < RPT)
            fidx = jnp.where(inr, rel, FJUNK)
            plsc.store_scatter(flags_v, [fidx], ones)
            return carry

        lax.fori_loop(0, BB // L, bscan, 0)

    def chunk_body(ci, carry):
        base = ci * CHUNK
        pltpu.sync_copy(src_hbm.at[pl.ds(base, CHUNK)], srcc_v)
        pltpu.sync_copy(dst_hbm.at[pl.ds(base, CHUNK)], dstc_v)

        def sel_body(g, cnt):
            vs = srcc_v[pl.ds(g * L, L)]
            vd = dstc_v[pl.ds(g * L, L)]
            rel = vd - lo
            m = (rel >= 0) & (rel < RPT)
            if filtered:
                fidx = jnp.where(m, rel, FJUNK)
                flt = plsc.load_gather(flags_v, [fidx])
                m = m & (flt > 0)
            cum = plsc.cumsum(m.astype(jnp.int32))
            pos = jnp.where(m, cnt + cum - 1, JUNK)
            plsc.store_scatter(selsrc_v, [pos], vs)
            plsc.store_scatter(seldst_v, [pos], rel)
            return cnt + cum[L - 1]

        cnt = lax.fori_loop(0, GROUPS, sel_body, jnp.int32(0))

        for t in range(NGRP):
            selsrc_v[pl.ds(cnt + t * L, L)] = jnp.zeros((L,), jnp.int32)
            seldst_v[pl.ds(cnt + t * L, L)] = jnp.full((L,), RPT, jnp.int32)
        ngroups = (cnt + (GB - 1)) // GB

        def fire(b, buf, sem):
            pltpu.async_copy(
                feat_hbm.at[selsrc_v.at[pl.ds(b * GB, GB)]], buf, sem)

        def wait(b, buf, sem):
            pltpu.make_async_copy(
                feat_hbm.at[selsrc_v.at[pl.ds(b * GB, GB)]], buf, sem).wait()

        def process(b, buf):
            for g in range(NGRP):
                ldv = seldst_v[pl.ds(b * GB + g * L, L)]
                for j in range(L):
                    ld = ldv[j]
                    for k in range(DD // L):
                        sl = pl.ds(k * L, L)
                        acc_v[ld, sl] = jnp.maximum(
                            acc_v[ld, sl], buf[g * L + j, sl])

        @pl.when(ngroups > 0)
        def _():
            fire(0, rows0_v, sem0)

        def grp_body(b, carry):
            par = lax.rem(b, 2)
            nxt = b + 1 < ngroups

            @pl.when(nxt & (par == 0))
            def _():
                fire(b + 1, rows1_v, sem1)

            @pl.when(nxt & (par == 1))
            def _():
                fire(b + 1, rows0_v, sem0)

            @pl.when(par == 0)
            def _():
                wait(b, rows0_v, sem0)
                process(b, rows0_v)

            @pl.when(par == 1)
            def _():
                wait(b, rows1_v, sem1)
                process(b, rows1_v)

            return carry

        lax.fori_loop(0, ngroups, grp_body, 0)
        return carry

    lax.fori_loop(0, EE // CHUNK, chunk_body, 0)

    def fin_row(r, carry):
        for k in range(DD // L):
            sl = pl.ds(k * L, L)
            v = acc_v[r, sl]
            acc_v[r, sl] = jnp.where(v == FMIN, 0.0, v)
        return carry

    lax.fori_loop(0, RPT, fin_row, 0)
    pltpu.sync_copy(acc_v.at[pl.ds(0, RPT)], out_hbm.at[pl.ds(lo, RPT)])


_SEG_SCRATCH = [
    pltpu.VMEM((ACC_ROWS, DD), jnp.float32),
    pltpu.VMEM((CHUNK,), jnp.int32),
    pltpu.VMEM((CHUNK,), jnp.int32),
    pltpu.VMEM((SELSZ,), jnp.int32),
    pltpu.VMEM((SELSZ,), jnp.int32),
    pltpu.VMEM((GB, DD), jnp.float32),
    pltpu.VMEM((GB, DD), jnp.float32),
    pltpu.VMEM((BB,), jnp.int32),
    pltpu.VMEM((FLAGSZ,), jnp.int32),
    pltpu.SemaphoreType.DMA,
    pltpu.SemaphoreType.DMA,
]

_segmax = pl.kernel(
    functools.partial(_segmax_impl, False),
    out_type=jax.ShapeDtypeStruct((NN, DD), jnp.float32),
    mesh=_mesh,
    compiler_params=pltpu.CompilerParams(needs_layout_passes=False),
    scratch_types=_SEG_SCRATCH,
)

_segmax_filt = pl.kernel(
    functools.partial(_segmax_impl, True),
    out_type=jax.ShapeDtypeStruct((NN, DD), jnp.float32),
    mesh=_mesh,
    compiler_params=pltpu.CompilerParams(needs_layout_passes=False),
    scratch_types=_SEG_SCRATCH,
)

BPW = BB // NW


def _gather2_body(h1_hbm, agg2_hbm, batch_hbm, outh_hbm, outa_hbm,
                  idx_v, rows_v, sem):
    wid = _worker_id()
    base = wid * BPW
    pltpu.sync_copy(batch_hbm.at[pl.ds(base, BPW)], idx_v)
    pltpu.async_copy(h1_hbm.at[idx_v], rows_v, sem).wait()
    pltpu.sync_copy(rows_v, outh_hbm.at[pl.ds(base, BPW)])
    pltpu.async_copy(agg2_hbm.at[idx_v], rows_v, sem).wait()
    pltpu.sync_copy(rows_v, outa_hbm.at[pl.ds(base, BPW)])


_gather2 = pl.kernel(
    _gather2_body,
    out_type=(
        jax.ShapeDtypeStruct((BB, DD), jnp.float32),
        jax.ShapeDtypeStruct((BB, DD), jnp.float32),
    ),
    mesh=_mesh,
    compiler_params=pltpu.CompilerParams(needs_layout_passes=False),
    scratch_types=[
        pltpu.VMEM((BPW,), jnp.int32),
        pltpu.VMEM((BPW, DD), jnp.float32),
        pltpu.SemaphoreType.DMA,
    ],
)


def _l1_body(x_ref, agg_ref, w1_ref, b1_ref, o_ref):
    w = w1_ref[...]
    h = lax.dot_general(x_ref[...], w[:, :DD], (((1,), (1,)), ((), ())),
                        preferred_element_type=jnp.float32)
    h += lax.dot_general(agg_ref[...], w[:, DD:], (((1,), (1,)), ((), ())),
                         preferred_element_type=jnp.float32)
    o_ref[...] = jnp.maximum(h + b1_ref[...], 0.0)


L1_BLK = 1000

_layer1 = pl.pallas_call(
    _l1_body,
    grid=(NN // L1_BLK,),
    in_specs=[
        pl.BlockSpec((L1_BLK, DD), lambda i: (i, 0)),
        pl.BlockSpec((L1_BLK, DD), lambda i: (i, 0)),
        pl.BlockSpec((DD, 2 * DD), lambda i: (0, 0)),
        pl.BlockSpec((1, DD), lambda i: (0, 0)),
    ],
    out_specs=pl.BlockSpec((L1_BLK, DD), lambda i: (i, 0)),
    out_shape=jax.ShapeDtypeStruct((NN, DD), jnp.float32),
)


def _l2_body(hb_ref, ab_ref, w2_ref, b2_ref, wo_ref, bo_ref, o_ref):
    wo = wo_ref[...]
    weff = lax.dot_general(wo, w2_ref[...], (((1,), (0,)), ((), ())),
                           preferred_element_type=jnp.float32)
    h = lax.dot_general(hb_ref[...], weff[:, :DD], (((1,), (1,)), ((), ())),
                        preferred_element_type=jnp.float32)
    h += lax.dot_general(ab_ref[...], weff[:, DD:], (((1,), (1,)), ((), ())),
                         preferred_element_type=jnp.float32)
    beff = lax.dot_general(b2_ref[...], wo, (((1,), (1,)), ((), ())),
                           preferred_element_type=jnp.float32)
    o_ref[...] = h + beff + bo_ref[...]


_layer2 = pl.pallas_call(
    _l2_body,
    out_shape=jax.ShapeDtypeStruct((BB, DD), jnp.float32),
)


@jax.jit
def kernel(x, edge_index, batch, W1, b1, W2, b2, Wout, bout):
    src = edge_index[0]
    dst = edge_index[1]
    agg1 = _segmax(x, src, dst, batch)
    h1 = _layer1(x, agg1, W1, b1.reshape(1, DD))
    agg2 = _segmax_filt(h1, src, dst, batch)
    h1b, a2b = _gather2(h1, agg2, batch)
    return _layer2(h1b, a2b, W2, b2.reshape(1, DD), Wout,
                   bout.reshape(1, DD))
